# initial kernel scaffold (unmeasured)
import jax
import jax.numpy as jnp
from jax import lax
from jax.experimental import pallas as pl
from jax.experimental.pallas import tpu as pltpu

N_DEV = 4
HQ = 8
DH = 128
SQ = 1024
SKV = 1024
DM = 1024
SCALE = 0.08838834764831843
NEG = -1e9


def _chunk_bias(c):
    qi = lax.broadcasted_iota(jnp.int32, (SQ, SKV), 0)
    ki = lax.broadcasted_iota(jnp.int32, (SQ, SKV), 1) + c * SKV
    local = jnp.abs(qi - ki) <= 128
    glob = (ki < 32) | (qi < 32)
    return jnp.where(local | glob, 0.0, NEG).astype(jnp.float32)


def _body(x_ref, wq_ref, wo_ref, kt_ref, vt_ref, out_ref,
          k_all, v_all, rs_buf, a2a_send, a2a_recv, copy_sems,
          ar_send, ar_recv):
    my = lax.axis_index("i")

    bar = pltpu.get_barrier_semaphore()
    for d in range(1, N_DEV):
        pl.semaphore_signal(bar, inc=1, device_id=((my + d) % N_DEV,),
                            device_id_type=pl.DeviceIdType.MESH)
    pl.semaphore_wait(bar, N_DEV - 1)

    local_cps = []
    for t, (src, dst) in enumerate(((kt_ref, k_all), (vt_ref, v_all))):
        cp = pltpu.make_async_copy(
            src.at[pl.ds(my * HQ, HQ)], dst.at[my], copy_sems.at[t])
        cp.start()
        local_cps.append(cp)

    sends = []
    for d in range(1, N_DEV):
        peer = (my + d) % N_DEV
        for t, (src, dst) in enumerate(((kt_ref, k_all), (vt_ref, v_all))):
            rdma = pltpu.make_async_remote_copy(
                src_ref=src.at[pl.ds(peer * HQ, HQ)],
                dst_ref=dst.at[my],
                send_sem=a2a_send.at[d - 1, t],
                recv_sem=a2a_recv.at[my, t],
                device_id=(peer,),
                device_id_type=pl.DeviceIdType.MESH,
            )
            rdma.start()
            sends.append(rdma)

    q = jnp.dot(x_ref[...], wq_ref[...], preferred_element_type=jnp.float32)
    q = (q * SCALE).astype(jnp.bfloat16)

    for cp in local_cps:
        cp.wait()
    for d in range(1, N_DEV):
        src_dev = (my - d) % N_DEV
        for t, (src, dst) in enumerate(((kt_ref, k_all), (vt_ref, v_all))):
            recv = pltpu.make_async_remote_copy(
                src_ref=src.at[pl.ds(0, HQ)],
                dst_ref=dst.at[src_dev],
                send_sem=a2a_send.at[d - 1, t],
                recv_sem=a2a_recv.at[src_dev, t],
                device_id=(my,),
                device_id_type=pl.DeviceIdType.MESH,
            )
            recv.wait_recv()

    accs = [jnp.zeros((SQ, DH), jnp.float32) for _ in range(HQ)]
    denoms = [jnp.zeros((SQ, 1), jnp.float32) for _ in range(HQ)]
    for c in range(N_DEV):
        bias = _chunk_bias(c)
        for h in range(HQ):
            qh = q[:, h * DH:(h + 1) * DH]
            s = lax.dot_general(qh, k_all[c, h], (((1,), (1,)), ((), ())),
                                preferred_element_type=jnp.float32)
            p = jnp.exp(s + bias)
            denoms[h] = denoms[h] + jnp.sum(p, axis=1, keepdims=True)
            accs[h] = accs[h] + lax.dot_general(
                p.astype(jnp.bfloat16), v_all[c, h],
                (((1,), (0,)), ((), ())), preferred_element_type=jnp.float32)

    out = None
    for h in range(HQ):
        ctx = (accs[h] / denoms[h]).astype(jnp.bfloat16)
        wo_h = wo_ref[h * DH:(h + 1) * DH, :].astype(jnp.bfloat16)
        po = jnp.dot(ctx, wo_h, preferred_element_type=jnp.float32)
        out = po if out is None else out + po

    for rdma in sends:
        rdma.wait_send()

    rs_buf[0, :, :] = out.astype(jnp.bfloat16)
    right = (my + 1) % N_DEV
    for h in range(N_DEV - 1):
        rdma = pltpu.make_async_remote_copy(
            src_ref=rs_buf.at[h],
            dst_ref=rs_buf.at[h + 1],
            send_sem=ar_send.at[h],
            recv_sem=ar_recv.at[h],
            device_id=(right,),
            device_id_type=pl.DeviceIdType.MESH,
        )
        rdma.start()
        rdma.wait()
        out = out + rs_buf[h + 1, :, :].astype(jnp.float32)
    out_ref[...] = out

    def _exit(sem):
        for d in range(1, N_DEV):
            pl.semaphore_signal(sem, inc=1, device_id=((my + d) % N_DEV,),
                                device_id_type=pl.DeviceIdType.MESH)
        pl.semaphore_wait(sem, N_DEV - 1)
    pl.run_scoped(_exit, sem=pltpu.SemaphoreType.REGULAR)


def kernel(x, Wq, K_ext, V_ext, Wo):
    kt = K_ext[0].transpose(1, 0, 2).astype(jnp.bfloat16)
    vt = V_ext[0].transpose(1, 0, 2).astype(jnp.bfloat16)

    out = pl.pallas_call(
        _body,
        out_shape=jax.ShapeDtypeStruct((SQ, DM), jnp.float32),
        in_specs=[
            pl.BlockSpec(memory_space=pltpu.VMEM),
            pl.BlockSpec(memory_space=pltpu.VMEM),
            pl.BlockSpec(memory_space=pltpu.VMEM),
            pl.BlockSpec(memory_space=pltpu.ANY),
            pl.BlockSpec(memory_space=pltpu.ANY),
        ],
        out_specs=pl.BlockSpec(memory_space=pltpu.VMEM),
        scratch_shapes=[
            pltpu.VMEM((N_DEV, HQ, SKV, DH), jnp.bfloat16),
            pltpu.VMEM((N_DEV, HQ, SKV, DH), jnp.bfloat16),
            pltpu.VMEM((N_DEV, SQ, DM), jnp.bfloat16),
            pltpu.SemaphoreType.DMA((N_DEV - 1, 2)),
            pltpu.SemaphoreType.DMA((N_DEV, 2)),
            pltpu.SemaphoreType.DMA((2,)),
            pltpu.SemaphoreType.DMA((N_DEV - 1,)),
            pltpu.SemaphoreType.DMA((N_DEV - 1,)),
        ],
        compiler_params=pltpu.CompilerParams(collective_id=0),
    )(x[0], Wq, Wo, kt, vt)
    return out[None]


# baseline (device time: 262586 ns/iter reference)
import jax
import jax.numpy as jnp
from jax import lax
from jax.experimental import pallas as pl
from jax.experimental.pallas import tpu as pltpu

N_DEV = 4
HQ = 8
DH = 128
SQ = 1024
SKV = 1024
DM = 1024
SCALE = 0.08838834764831843
NEG = -1e9


def _chunk_bias(c):
    qi = lax.broadcasted_iota(jnp.int32, (SQ, SKV), 0)
    ki = lax.broadcasted_iota(jnp.int32, (SQ, SKV), 1) + c * SKV
    local = jnp.abs(qi - ki) <= 128
    glob = (ki < 32) | (qi < 32)
    return jnp.where(local | glob, 0.0, NEG).astype(jnp.float32)


def _body(q_ref, wo_ref, kt_ref, vt_ref, out_ref,
          k_all, v_all, rs_buf, a2a_send, a2a_recv, copy_sems,
          ar_send, ar_recv):
    my = lax.axis_index("i")

    bar = pltpu.get_barrier_semaphore()
    for d in range(1, N_DEV):
        pl.semaphore_signal(bar, inc=1, device_id=((my + d) % N_DEV,),
                            device_id_type=pl.DeviceIdType.MESH)
    pl.semaphore_wait(bar, N_DEV - 1)

    local_cps = []
    for t, (src, dst) in enumerate(((kt_ref, k_all), (vt_ref, v_all))):
        cp = pltpu.make_async_copy(
            src.at[pl.ds(my * HQ, HQ)], dst.at[my], copy_sems.at[t])
        cp.start()
        local_cps.append(cp)

    sends = []
    for d in range(1, N_DEV):
        peer = (my + d) % N_DEV
        for t, (src, dst) in enumerate(((kt_ref, k_all), (vt_ref, v_all))):
            rdma = pltpu.make_async_remote_copy(
                src_ref=src.at[pl.ds(peer * HQ, HQ)],
                dst_ref=dst.at[my],
                send_sem=a2a_send.at[d - 1, t],
                recv_sem=a2a_recv.at[my, t],
                device_id=(peer,),
                device_id_type=pl.DeviceIdType.MESH,
            )
            rdma.start()
            sends.append(rdma)

    for cp in local_cps:
        cp.wait()
    for d in range(1, N_DEV):
        src_dev = (my - d) % N_DEV
        for t, (src, dst) in enumerate(((kt_ref, k_all), (vt_ref, v_all))):
            recv = pltpu.make_async_remote_copy(
                src_ref=src.at[pl.ds(0, HQ)],
                dst_ref=dst.at[src_dev],
                send_sem=a2a_send.at[d - 1, t],
                recv_sem=a2a_recv.at[src_dev, t],
                device_id=(my,),
                device_id_type=pl.DeviceIdType.MESH,
            )
            recv.wait_recv()

    for h in range(HQ):
        qh = q_ref[:, h * DH:(h + 1) * DH]
        acc = jnp.zeros((SQ, DH), jnp.float32)
        den = jnp.zeros((SQ, 1), jnp.float32)
        for c in range(N_DEV):
            s = lax.dot_general(qh, k_all[c, h], (((1,), (1,)), ((), ())),
                                preferred_element_type=jnp.float32)
            p = jnp.exp(s + _chunk_bias(c))
            den = den + jnp.sum(p, axis=1, keepdims=True)
            acc = acc + lax.dot_general(
                p.astype(jnp.bfloat16), v_all[c, h],
                (((1,), (0,)), ((), ())), preferred_element_type=jnp.float32)
        ctx = (acc / den).astype(jnp.bfloat16)
        wo_h = wo_ref[h * DH:(h + 1) * DH, :].astype(jnp.bfloat16)
        po = jnp.dot(ctx, wo_h, preferred_element_type=jnp.float32)
        if h == 0:
            out_ref[...] = po
        else:
            out_ref[...] = out_ref[...] + po

    for rdma in sends:
        rdma.wait_send()

    rs_buf[0, :, :] = out_ref[...].astype(jnp.bfloat16)
    right = (my + 1) % N_DEV
    for h in range(N_DEV - 1):
        rdma = pltpu.make_async_remote_copy(
            src_ref=rs_buf.at[h % 2],
            dst_ref=rs_buf.at[(h + 1) % 2],
            send_sem=ar_send.at[h],
            recv_sem=ar_recv.at[h],
            device_id=(right,),
            device_id_type=pl.DeviceIdType.MESH,
        )
        rdma.start()
        rdma.wait()
        out_ref[...] = out_ref[...] + rs_buf[(h + 1) % 2, :, :].astype(jnp.float32)

    def _exit(sem):
        for d in range(1, N_DEV):
            pl.semaphore_signal(sem, inc=1, device_id=((my + d) % N_DEV,),
                                device_id_type=pl.DeviceIdType.MESH)
        pl.semaphore_wait(sem, N_DEV - 1)
    pl.run_scoped(_exit, sem=pltpu.SemaphoreType.REGULAR)


def kernel(x, Wq, K_ext, V_ext, Wo):
    kt = K_ext[0].transpose(1, 0, 2).astype(jnp.bfloat16)
    vt = V_ext[0].transpose(1, 0, 2).astype(jnp.bfloat16)
    q = (jnp.dot(x[0], Wq, preferred_element_type=jnp.float32)
         * SCALE).astype(jnp.bfloat16)

    out = pl.pallas_call(
        _body,
        out_shape=jax.ShapeDtypeStruct((SQ, DM), jnp.float32),
        in_specs=[
            pl.BlockSpec(memory_space=pltpu.VMEM),
            pl.BlockSpec(memory_space=pltpu.VMEM),
            pl.BlockSpec(memory_space=pl.ANY),
            pl.BlockSpec(memory_space=pl.ANY),
        ],
        out_specs=pl.BlockSpec(memory_space=pltpu.VMEM),
        scratch_shapes=[
            pltpu.VMEM((N_DEV, HQ, SKV, DH), jnp.bfloat16),
            pltpu.VMEM((N_DEV, HQ, SKV, DH), jnp.bfloat16),
            pltpu.VMEM((2, SQ, DM), jnp.bfloat16),
            pltpu.SemaphoreType.DMA((N_DEV - 1, 2)),
            pltpu.SemaphoreType.DMA((N_DEV, 2)),
            pltpu.SemaphoreType.DMA((2,)),
            pltpu.SemaphoreType.DMA((N_DEV - 1,)),
            pltpu.SemaphoreType.DMA((N_DEV - 1,)),
        ],
        compiler_params=pltpu.CompilerParams(
            collective_id=0, vmem_limit_bytes=50 * 1024 * 1024),
    )(q, Wo, kt, vt)
    return out[None]


# device time: 189828 ns/iter; 1.3833x vs baseline; 1.3833x over previous
import jax
import jax.numpy as jnp
from jax import lax
from jax.experimental import pallas as pl
from jax.experimental.pallas import tpu as pltpu

N_DEV = 4
HQ = 8
HQ_TOT = 32
DH = 128
SQ = 1024
SKV = 1024
DM = 1024
NG = 32
NB1 = 128
SCALE = 0.08838834764831843
NEG = -1e9

_MESH = pl.DeviceIdType.MESH


def _body(q_ref, wo_ref, kt_ref, vt_ref, out_ref,
          k0_buf, v0_buf, k1_buf, v1_buf, relay_buf, qg_buf,
          ga_send, gd_send, ga_recv, gd_recv, sbuf, rbuf,
          k0_send, relay_send, fwd_send, relay_recv, k0_recv,
          k1_send, k1_recv, qg_send_s, qg_recv_s, ga_send_s, ga_recv_s,
          cp_sem, bf_send, bf_recv):
    my = lax.axis_index("i")

    bar = pltpu.get_barrier_semaphore()
    for d in range(1, N_DEV):
        pl.semaphore_signal(bar, inc=1, device_id=((my + d) % N_DEV,),
                            device_id_type=_MESH)
    pl.semaphore_wait(bar, N_DEV - 1)

    kv_pairs = ((kt_ref, k0_buf), (vt_ref, v0_buf))

    @pl.when(my == 0)
    def _():
        pltpu.make_async_copy(kt_ref.at[pl.ds(0, HQ)], k0_buf,
                              cp_sem.at[0]).start()
        pltpu.make_async_copy(vt_ref.at[pl.ds(0, HQ)], v0_buf,
                              cp_sem.at[1]).start()

    @pl.when(my == 1)
    def _():
        pltpu.make_async_copy(kt_ref.at[pl.ds(HQ, HQ), pl.ds(0, NB1)],
                              k1_buf, cp_sem.at[2]).start()
        pltpu.make_async_copy(vt_ref.at[pl.ds(HQ, HQ), pl.ds(0, NB1)],
                              v1_buf, cp_sem.at[3]).start()

    pltpu.make_async_copy(q_ref.at[pl.ds(0, NG)], qg_buf.at[my],
                          cp_sem.at[4]).start()

    for d in range(1, N_DEV):
        peer = (my + d) % N_DEV
        pltpu.make_async_remote_copy(
            src_ref=q_ref.at[pl.ds(0, NG)], dst_ref=qg_buf.at[my],
            send_sem=qg_send_s.at[d - 1], recv_sem=qg_recv_s.at[my],
            device_id=(peer,), device_id_type=_MESH).start()

    @pl.when(my == 0)
    def _():
        diag = (my + 2) % N_DEV
        pltpu.make_async_remote_copy(
            src_ref=kt_ref.at[pl.ds(diag * HQ, HQ)], dst_ref=relay_buf,
            send_sem=relay_send.at[0], recv_sem=relay_recv.at[0],
            device_id=((my + 1) % N_DEV,), device_id_type=_MESH).start()
        pltpu.make_async_remote_copy(
            src_ref=vt_ref.at[pl.ds(diag * HQ, HQ)], dst_ref=relay_buf,
            send_sem=relay_send.at[1], recv_sem=relay_recv.at[1],
            device_id=((my + 3) % N_DEV,), device_id_type=_MESH).start()

    @pl.when(my == 1)
    def _():
        for d in range(1, N_DEV):
            peer = (my + d) % N_DEV
            for t, (src, dst) in enumerate(((kt_ref, k1_buf),
                                            (vt_ref, v1_buf))):
                pltpu.make_async_remote_copy(
                    src_ref=src.at[pl.ds(peer * HQ, HQ), pl.ds(0, NB1)],
                    dst_ref=dst,
                    send_sem=k1_send.at[d - 1, t], recv_sem=k1_recv.at[t],
                    device_id=(peer,), device_id_type=_MESH).start()

    @pl.when(my == 0)
    def _():
        diag = (my + 2) % N_DEV
        pltpu.make_async_remote_copy(
            src_ref=kt_ref.at[pl.ds(diag * HQ, HQ)], dst_ref=relay_buf,
            send_sem=relay_send.at[0], recv_sem=relay_recv.at[0],
            device_id=((my + 1) % N_DEV,), device_id_type=_MESH).wait_send()
        pltpu.make_async_remote_copy(
            src_ref=vt_ref.at[pl.ds(diag * HQ, HQ)], dst_ref=relay_buf,
            send_sem=relay_send.at[1], recv_sem=relay_recv.at[1],
            device_id=((my + 3) % N_DEV,), device_id_type=_MESH).wait_send()
        for i, d in enumerate((1, 3)):
            peer = (my + d) % N_DEV
            for t, (src, dst) in enumerate(kv_pairs):
                pltpu.make_async_remote_copy(
                    src_ref=src.at[pl.ds(peer * HQ, HQ)], dst_ref=dst,
                    send_sem=k0_send.at[i, t], recv_sem=k0_recv.at[t],
                    device_id=(peer,), device_id_type=_MESH).start()

    @pl.when(my == 1)
    def _():
        pltpu.make_async_remote_copy(
            src_ref=kt_ref.at[pl.ds(0, HQ)], dst_ref=relay_buf,
            send_sem=relay_send.at[0], recv_sem=relay_recv.at[0],
            device_id=(my,), device_id_type=_MESH).wait_recv()
        pltpu.make_async_remote_copy(
            src_ref=relay_buf, dst_ref=k0_buf,
            send_sem=fwd_send.at[0], recv_sem=k0_recv.at[0],
            device_id=((my + 1) % N_DEV,), device_id_type=_MESH).start()

    @pl.when(my == 3)
    def _():
        pltpu.make_async_remote_copy(
            src_ref=vt_ref.at[pl.ds(0, HQ)], dst_ref=relay_buf,
            send_sem=relay_send.at[1], recv_sem=relay_recv.at[1],
            device_id=(my,), device_id_type=_MESH).wait_recv()
        pltpu.make_async_remote_copy(
            src_ref=relay_buf, dst_ref=v0_buf,
            send_sem=fwd_send.at[1], recv_sem=k0_recv.at[1],
            device_id=((my - 1) % N_DEV,), device_id_type=_MESH).start()

    pltpu.make_async_copy(q_ref.at[pl.ds(0, NG)], qg_buf.at[my],
                          cp_sem.at[4]).wait()
    for d in range(N_DEV):
        src_dev = (my - d) % N_DEV
        if d > 0:
            pltpu.make_async_remote_copy(
                src_ref=q_ref.at[pl.ds(0, NG)], dst_ref=qg_buf.at[src_dev],
                send_sem=qg_send_s.at[0], recv_sem=qg_recv_s.at[src_dev],
                device_id=(my,), device_id_type=_MESH).wait_recv()

    for j in range(N_DEV):
        for h in range(HQ):
            H = j * HQ + h
            qg = qg_buf[j, :, h * DH:(h + 1) * DH]
            s = lax.dot_general(qg, kt_ref[H], (((1,), (1,)), ((), ())),
                                preferred_element_type=jnp.float32)
            p = jnp.exp(s)
            gd_send[j, h, :, :] = jnp.sum(p, axis=1, keepdims=True)
            ga_send[j, h, :, :] = lax.dot_general(
                p.astype(jnp.bfloat16), vt_ref[H],
                (((1,), (0,)), ((), ())),
                preferred_element_type=jnp.float32).astype(jnp.bfloat16)

    pltpu.make_async_copy(ga_send.at[my], ga_recv.at[my], cp_sem.at[5]).start()
    pltpu.make_async_copy(gd_send.at[my], gd_recv.at[my], cp_sem.at[6]).start()
    for d in range(1, N_DEV):
        peer = (my + d) % N_DEV
        for t, (src, dst) in enumerate(((ga_send, ga_recv),
                                        (gd_send, gd_recv))):
            pltpu.make_async_remote_copy(
                src_ref=src.at[peer], dst_ref=dst.at[my],
                send_sem=ga_send_s.at[d - 1, t], recv_sem=ga_recv_s.at[my, t],
                device_id=(peer,), device_id_type=_MESH).start()

    @pl.when(my == 0)
    def _():
        pltpu.make_async_copy(kt_ref.at[pl.ds(0, HQ)], k0_buf,
                              cp_sem.at[0]).wait()
        pltpu.make_async_copy(vt_ref.at[pl.ds(0, HQ)], v0_buf,
                              cp_sem.at[1]).wait()

    @pl.when(my != 0)
    def _():
        for t, (src, dst) in enumerate(kv_pairs):
            pltpu.make_async_remote_copy(
                src_ref=src.at[pl.ds(0, HQ)], dst_ref=dst,
                send_sem=k0_send.at[0, t], recv_sem=k0_recv.at[t],
                device_id=(my,), device_id_type=_MESH).wait_recv()

    @pl.when(my == 1)
    def _():
        pltpu.make_async_copy(kt_ref.at[pl.ds(HQ, HQ), pl.ds(0, NB1)],
                              k1_buf, cp_sem.at[2]).wait()
        pltpu.make_async_copy(vt_ref.at[pl.ds(HQ, HQ), pl.ds(0, NB1)],
                              v1_buf, cp_sem.at[3]).wait()

    @pl.when(my != 1)
    def _():
        for t, (src, dst) in enumerate(((kt_ref, k1_buf), (vt_ref, v1_buf))):
            pltpu.make_async_remote_copy(
                src_ref=src.at[pl.ds(0, HQ), pl.ds(0, NB1)], dst_ref=dst,
                send_sem=k1_send.at[0, t], recv_sem=k1_recv.at[t],
                device_id=(my,), device_id_type=_MESH).wait_recv()

    qi = lax.broadcasted_iota(jnp.int32, (SQ, SKV), 0)
    ki = lax.broadcasted_iota(jnp.int32, (SQ, SKV), 1)
    mask0 = (jnp.abs(qi - ki) <= 128) | (ki < 32) | (qi < 32)
    bias0 = jnp.where(mask0, 0.0, NEG).astype(jnp.float32)
    qi1 = lax.broadcasted_iota(jnp.int32, (SQ, NB1), 0)
    ki1 = lax.broadcasted_iota(jnp.int32, (SQ, NB1), 1)
    bias1 = jnp.where(qi1 >= 896 + ki1, 0.0, NEG).astype(jnp.float32)

    for h in range(HQ):
        qh = q_ref[:, h * DH:(h + 1) * DH]
        s0 = lax.dot_general(qh, k0_buf[h], (((1,), (1,)), ((), ())),
                             preferred_element_type=jnp.float32)
        p0 = jnp.exp(s0 + bias0)
        den = jnp.sum(p0, axis=1, keepdims=True)
        acc = lax.dot_general(p0.astype(jnp.bfloat16), v0_buf[h],
                              (((1,), (0,)), ((), ())),
                              preferred_element_type=jnp.float32)
        s1 = lax.dot_general(qh, k1_buf[h], (((1,), (1,)), ((), ())),
                             preferred_element_type=jnp.float32)
        p1 = jnp.exp(s1 + bias1)
        den = den + jnp.sum(p1, axis=1, keepdims=True)
        acc = acc + lax.dot_general(p1.astype(jnp.bfloat16), v1_buf[h],
                                    (((1,), (0,)), ((), ())),
                                    preferred_element_type=jnp.float32)
        band_ctx = ((acc / den)[32:, :]).astype(jnp.bfloat16)
        po = jnp.dot(band_ctx, wo_ref[h * DH:(h + 1) * DH, :],
                     preferred_element_type=jnp.float32)
        if h == 0:
            out_ref[32:, :] = po
        else:
            out_ref[32:, :] = out_ref[32:, :] + po

    pltpu.make_async_copy(ga_send.at[my], ga_recv.at[my], cp_sem.at[5]).wait()
    pltpu.make_async_copy(gd_send.at[my], gd_recv.at[my], cp_sem.at[6]).wait()
    for d in range(1, N_DEV):
        src_dev = (my - d) % N_DEV
        for t, (src, dst) in enumerate(((ga_send, ga_recv),
                                        (gd_send, gd_recv))):
            pltpu.make_async_remote_copy(
                src_ref=src.at[src_dev], dst_ref=dst.at[src_dev],
                send_sem=ga_send_s.at[0, t], recv_sem=ga_recv_s.at[src_dev, t],
                device_id=(my,), device_id_type=_MESH).wait_recv()

    for h in range(HQ):
        acc = jnp.zeros((NG, DH), jnp.float32)
        den = jnp.zeros((NG, 1), jnp.float32)
        for j in range(N_DEV):
            acc = acc + ga_recv[j, h].astype(jnp.float32)
            den = den + gd_recv[j, h]
        gctx = (acc / den).astype(jnp.bfloat16)
        po = jnp.dot(gctx, wo_ref[h * DH:(h + 1) * DH, :],
                     preferred_element_type=jnp.float32)
        if h == 0:
            out_ref[0:32, :] = po
        else:
            out_ref[0:32, :] = out_ref[0:32, :] + po

    for stage, step in enumerate((1, 2)):
        partner = jnp.bitwise_xor(my, step)
        sbuf[stage, :, :] = out_ref[...].astype(jnp.bfloat16)
        ex = pltpu.make_async_remote_copy(
            src_ref=sbuf.at[stage], dst_ref=rbuf.at[stage],
            send_sem=bf_send.at[stage], recv_sem=bf_recv.at[stage],
            device_id=(partner,), device_id_type=_MESH)
        ex.start()
        ex.wait()
        out_ref[...] = out_ref[...] + rbuf[stage, :, :].astype(jnp.float32)

    for d in range(1, N_DEV):
        peer = (my + d) % N_DEV
        pltpu.make_async_remote_copy(
            src_ref=q_ref.at[pl.ds(0, NG)], dst_ref=qg_buf.at[my],
            send_sem=qg_send_s.at[d - 1], recv_sem=qg_recv_s.at[my],
            device_id=(peer,), device_id_type=_MESH).wait_send()
        for t, (src, dst) in enumerate(((ga_send, ga_recv),
                                        (gd_send, gd_recv))):
            pltpu.make_async_remote_copy(
                src_ref=src.at[peer], dst_ref=dst.at[my],
                send_sem=ga_send_s.at[d - 1, t], recv_sem=ga_recv_s.at[my, t],
                device_id=(peer,), device_id_type=_MESH).wait_send()

    @pl.when(my == 0)
    def _():
        for i, d in enumerate((1, 3)):
            peer = (my + d) % N_DEV
            for t, (src, dst) in enumerate(kv_pairs):
                pltpu.make_async_remote_copy(
                    src_ref=src.at[pl.ds(peer * HQ, HQ)], dst_ref=dst,
                    send_sem=k0_send.at[i, t], recv_sem=k0_recv.at[t],
                    device_id=(peer,), device_id_type=_MESH).wait_send()

    @pl.when(my == 1)
    def _():
        for d in range(1, N_DEV):
            peer = (my + d) % N_DEV
            for t, (src, dst) in enumerate(((kt_ref, k1_buf),
                                            (vt_ref, v1_buf))):
                pltpu.make_async_remote_copy(
                    src_ref=src.at[pl.ds(peer * HQ, HQ), pl.ds(0, NB1)],
                    dst_ref=dst,
                    send_sem=k1_send.at[d - 1, t], recv_sem=k1_recv.at[t],
                    device_id=(peer,), device_id_type=_MESH).wait_send()
        pltpu.make_async_remote_copy(
            src_ref=relay_buf, dst_ref=k0_buf,
            send_sem=fwd_send.at[0], recv_sem=k0_recv.at[0],
            device_id=((my + 1) % N_DEV,), device_id_type=_MESH).wait_send()

    @pl.when(my == 3)
    def _():
        pltpu.make_async_remote_copy(
            src_ref=relay_buf, dst_ref=v0_buf,
            send_sem=fwd_send.at[1], recv_sem=k0_recv.at[1],
            device_id=((my - 1) % N_DEV,), device_id_type=_MESH).wait_send()

    def _exit(sem):
        for d in range(1, N_DEV):
            pl.semaphore_signal(sem, inc=1, device_id=((my + d) % N_DEV,),
                                device_id_type=_MESH)
        pl.semaphore_wait(sem, N_DEV - 1)
    pl.run_scoped(_exit, sem=pltpu.SemaphoreType.REGULAR)


def kernel(x, Wq, K_ext, V_ext, Wo):
    kt = K_ext[0].transpose(1, 0, 2).astype(jnp.bfloat16)
    vt = V_ext[0].transpose(1, 0, 2).astype(jnp.bfloat16)
    q = (jnp.dot(x[0], Wq, preferred_element_type=jnp.float32)
         * SCALE).astype(jnp.bfloat16)
    wo = Wo.astype(jnp.bfloat16)

    out = pl.pallas_call(
        _body,
        out_shape=jax.ShapeDtypeStruct((SQ, DM), jnp.float32),
        in_specs=[
            pl.BlockSpec(memory_space=pltpu.VMEM),
            pl.BlockSpec(memory_space=pltpu.VMEM),
            pl.BlockSpec(memory_space=pltpu.VMEM),
            pl.BlockSpec(memory_space=pltpu.VMEM),
        ],
        out_specs=pl.BlockSpec(memory_space=pltpu.VMEM),
        scratch_shapes=[
            pltpu.VMEM((HQ, SKV, DH), jnp.bfloat16),
            pltpu.VMEM((HQ, SKV, DH), jnp.bfloat16),
            pltpu.VMEM((HQ, NB1, DH), jnp.bfloat16),
            pltpu.VMEM((HQ, NB1, DH), jnp.bfloat16),
            pltpu.VMEM((HQ, SKV, DH), jnp.bfloat16),
            pltpu.VMEM((N_DEV, NG, DM), jnp.bfloat16),
            pltpu.VMEM((N_DEV, HQ, NG, DH), jnp.bfloat16),
            pltpu.VMEM((N_DEV, HQ, NG, 1), jnp.float32),
            pltpu.VMEM((N_DEV, HQ, NG, DH), jnp.bfloat16),
            pltpu.VMEM((N_DEV, HQ, NG, 1), jnp.float32),
            pltpu.VMEM((2, SQ, DM), jnp.bfloat16),
            pltpu.VMEM((2, SQ, DM), jnp.bfloat16),
            pltpu.SemaphoreType.DMA((2, 2)),
            pltpu.SemaphoreType.DMA((2,)),
            pltpu.SemaphoreType.DMA((2,)),
            pltpu.SemaphoreType.DMA((2,)),
            pltpu.SemaphoreType.DMA((2,)),
            pltpu.SemaphoreType.DMA((N_DEV - 1, 2)),
            pltpu.SemaphoreType.DMA((2,)),
            pltpu.SemaphoreType.DMA((N_DEV - 1,)),
            pltpu.SemaphoreType.DMA((N_DEV,)),
            pltpu.SemaphoreType.DMA((N_DEV - 1, 2)),
            pltpu.SemaphoreType.DMA((N_DEV, 2)),
            pltpu.SemaphoreType.DMA((7,)),
            pltpu.SemaphoreType.DMA((2,)),
            pltpu.SemaphoreType.DMA((2,)),
        ],
        compiler_params=pltpu.CompilerParams(
            collective_id=0, vmem_limit_bytes=46 * 1024 * 1024),
    )(q, wo, kt, vt)
    return out[None]


# device time: 179353 ns/iter; 1.4641x vs baseline; 1.0584x over previous
import jax
import jax.numpy as jnp
from jax import lax
from jax.experimental import pallas as pl
from jax.experimental.pallas import tpu as pltpu

N_DEV = 4
HQ = 8
HQ_TOT = 32
DH = 128
SQ = 1024
SKV = 1024
DM = 1024
NG = 32
NB1 = 128
SCALE = 0.08838834764831843
NEG = -1e9
HALF = 512

_MESH = pl.DeviceIdType.MESH


def _body(q_ref, wo_ref, kt_ref, vt_ref, out_ref,
          k0_buf, v0_buf, k1_buf, v1_buf, relay_buf, qg_buf,
          ga_send, gd_send, ga_recv, gd_recv, sbuf, rbuf,
          k0_send, relay_send, fwd_send, relay_recv, k0_recv,
          k1_send, k1_recv, qg_send_s, qg_recv_s, ga_send_s, ga_recv_s,
          cp_sem, bf_send, bf_recv):
    my = lax.axis_index("i")

    bar = pltpu.get_barrier_semaphore()
    for d in range(1, N_DEV):
        pl.semaphore_signal(bar, inc=1, device_id=((my + d) % N_DEV,),
                            device_id_type=_MESH)
    pl.semaphore_wait(bar, N_DEV - 1)

    kv_pairs = ((kt_ref, k0_buf), (vt_ref, v0_buf))

    @pl.when(my == 0)
    def _():
        pltpu.make_async_copy(kt_ref.at[pl.ds(0, HQ)], k0_buf,
                              cp_sem.at[0]).start()
        pltpu.make_async_copy(vt_ref.at[pl.ds(0, HQ)], v0_buf,
                              cp_sem.at[1]).start()

    @pl.when(my == 1)
    def _():
        pltpu.make_async_copy(kt_ref.at[pl.ds(HQ, HQ), pl.ds(0, NB1)],
                              k1_buf, cp_sem.at[2]).start()
        pltpu.make_async_copy(vt_ref.at[pl.ds(HQ, HQ), pl.ds(0, NB1)],
                              v1_buf, cp_sem.at[3]).start()

    pltpu.make_async_copy(q_ref.at[pl.ds(0, NG)], qg_buf.at[my],
                          cp_sem.at[4]).start()

    for d in range(1, N_DEV):
        peer = (my + d) % N_DEV
        pltpu.make_async_remote_copy(
            src_ref=q_ref.at[pl.ds(0, NG)], dst_ref=qg_buf.at[my],
            send_sem=qg_send_s.at[d - 1], recv_sem=qg_recv_s.at[my],
            device_id=(peer,), device_id_type=_MESH).start()

    @pl.when(my == 0)
    def _():
        diag = (my + 2) % N_DEV
        pltpu.make_async_remote_copy(
            src_ref=kt_ref.at[pl.ds(diag * HQ, HQ)], dst_ref=relay_buf,
            send_sem=relay_send.at[0], recv_sem=relay_recv.at[0],
            device_id=((my + 1) % N_DEV,), device_id_type=_MESH).start()
        pltpu.make_async_remote_copy(
            src_ref=vt_ref.at[pl.ds(diag * HQ, HQ)], dst_ref=relay_buf,
            send_sem=relay_send.at[1], recv_sem=relay_recv.at[1],
            device_id=((my + 3) % N_DEV,), device_id_type=_MESH).start()

    @pl.when(my == 1)
    def _():
        for d in range(1, N_DEV):
            peer = (my + d) % N_DEV
            for t, (src, dst) in enumerate(((kt_ref, k1_buf),
                                            (vt_ref, v1_buf))):
                pltpu.make_async_remote_copy(
                    src_ref=src.at[pl.ds(peer * HQ, HQ), pl.ds(0, NB1)],
                    dst_ref=dst,
                    send_sem=k1_send.at[d - 1, t], recv_sem=k1_recv.at[t],
                    device_id=(peer,), device_id_type=_MESH).start()

    @pl.when(my == 0)
    def _():
        diag = (my + 2) % N_DEV
        pltpu.make_async_remote_copy(
            src_ref=kt_ref.at[pl.ds(diag * HQ, HQ)], dst_ref=relay_buf,
            send_sem=relay_send.at[0], recv_sem=relay_recv.at[0],
            device_id=((my + 1) % N_DEV,), device_id_type=_MESH).wait_send()
        pltpu.make_async_remote_copy(
            src_ref=vt_ref.at[pl.ds(diag * HQ, HQ)], dst_ref=relay_buf,
            send_sem=relay_send.at[1], recv_sem=relay_recv.at[1],
            device_id=((my + 3) % N_DEV,), device_id_type=_MESH).wait_send()
        for i, d in enumerate((1, 3)):
            peer = (my + d) % N_DEV
            for t, (src, dst) in enumerate(kv_pairs):
                pltpu.make_async_remote_copy(
                    src_ref=src.at[pl.ds(peer * HQ, HQ)], dst_ref=dst,
                    send_sem=k0_send.at[i, t], recv_sem=k0_recv.at[t],
                    device_id=(peer,), device_id_type=_MESH).start()

    @pl.when(my == 1)
    def _():
        pltpu.make_async_remote_copy(
            src_ref=kt_ref.at[pl.ds(0, HQ)], dst_ref=relay_buf,
            send_sem=relay_send.at[0], recv_sem=relay_recv.at[0],
            device_id=(my,), device_id_type=_MESH).wait_recv()
        pltpu.make_async_remote_copy(
            src_ref=relay_buf, dst_ref=k0_buf,
            send_sem=fwd_send.at[0], recv_sem=k0_recv.at[0],
            device_id=((my + 1) % N_DEV,), device_id_type=_MESH).start()

    @pl.when(my == 3)
    def _():
        pltpu.make_async_remote_copy(
            src_ref=vt_ref.at[pl.ds(0, HQ)], dst_ref=relay_buf,
            send_sem=relay_send.at[1], recv_sem=relay_recv.at[1],
            device_id=(my,), device_id_type=_MESH).wait_recv()
        pltpu.make_async_remote_copy(
            src_ref=relay_buf, dst_ref=v0_buf,
            send_sem=fwd_send.at[1], recv_sem=k0_recv.at[1],
            device_id=((my - 1) % N_DEV,), device_id_type=_MESH).start()

    pltpu.make_async_copy(q_ref.at[pl.ds(0, NG)], qg_buf.at[my],
                          cp_sem.at[4]).wait()
    for d in range(N_DEV):
        src_dev = (my - d) % N_DEV
        if d > 0:
            pltpu.make_async_remote_copy(
                src_ref=q_ref.at[pl.ds(0, NG)], dst_ref=qg_buf.at[src_dev],
                send_sem=qg_send_s.at[0], recv_sem=qg_recv_s.at[src_dev],
                device_id=(my,), device_id_type=_MESH).wait_recv()

    for j in range(N_DEV):
        for h in range(HQ):
            H = j * HQ + h
            qg = qg_buf[j, :, h * DH:(h + 1) * DH]
            s = lax.dot_general(qg, kt_ref[H], (((1,), (1,)), ((), ())),
                                preferred_element_type=jnp.float32)
            p = jnp.exp(s)
            gd_send[j, h, :, :] = jnp.sum(p, axis=1, keepdims=True)
            ga_send[j, h, :, :] = lax.dot_general(
                p.astype(jnp.bfloat16), vt_ref[H],
                (((1,), (0,)), ((), ())),
                preferred_element_type=jnp.float32).astype(jnp.bfloat16)

    pltpu.make_async_copy(ga_send.at[my], ga_recv.at[my], cp_sem.at[5]).start()
    pltpu.make_async_copy(gd_send.at[my], gd_recv.at[my], cp_sem.at[6]).start()
    for d in range(1, N_DEV):
        peer = (my + d) % N_DEV
        for t, (src, dst) in enumerate(((ga_send, ga_recv),
                                        (gd_send, gd_recv))):
            pltpu.make_async_remote_copy(
                src_ref=src.at[peer], dst_ref=dst.at[my],
                send_sem=ga_send_s.at[d - 1, t], recv_sem=ga_recv_s.at[my, t],
                device_id=(peer,), device_id_type=_MESH).start()

    @pl.when(my == 0)
    def _():
        pltpu.make_async_copy(kt_ref.at[pl.ds(0, HQ)], k0_buf,
                              cp_sem.at[0]).wait()
        pltpu.make_async_copy(vt_ref.at[pl.ds(0, HQ)], v0_buf,
                              cp_sem.at[1]).wait()

    @pl.when(my != 0)
    def _():
        for t, (src, dst) in enumerate(kv_pairs):
            pltpu.make_async_remote_copy(
                src_ref=src.at[pl.ds(0, HQ)], dst_ref=dst,
                send_sem=k0_send.at[0, t], recv_sem=k0_recv.at[t],
                device_id=(my,), device_id_type=_MESH).wait_recv()

    @pl.when(my == 1)
    def _():
        pltpu.make_async_copy(kt_ref.at[pl.ds(HQ, HQ), pl.ds(0, NB1)],
                              k1_buf, cp_sem.at[2]).wait()
        pltpu.make_async_copy(vt_ref.at[pl.ds(HQ, HQ), pl.ds(0, NB1)],
                              v1_buf, cp_sem.at[3]).wait()

    @pl.when(my != 1)
    def _():
        for t, (src, dst) in enumerate(((kt_ref, k1_buf), (vt_ref, v1_buf))):
            pltpu.make_async_remote_copy(
                src_ref=src.at[pl.ds(0, HQ), pl.ds(0, NB1)], dst_ref=dst,
                send_sem=k1_send.at[0, t], recv_sem=k1_recv.at[t],
                device_id=(my,), device_id_type=_MESH).wait_recv()

    qi = lax.broadcasted_iota(jnp.int32, (SQ, SKV), 0)
    ki = lax.broadcasted_iota(jnp.int32, (SQ, SKV), 1)
    mask0 = (jnp.abs(qi - ki) <= 128) | (ki < 32) | (qi < 32)
    bias0 = jnp.where(mask0, 0.0, NEG).astype(jnp.float32)
    qi1 = lax.broadcasted_iota(jnp.int32, (SQ, NB1), 0)
    ki1 = lax.broadcasted_iota(jnp.int32, (SQ, NB1), 1)
    bias1 = jnp.where(qi1 >= 896 + ki1, 0.0, NEG).astype(jnp.float32)

    def _band_pass(r0, nr):
        for h in range(HQ):
            qh = q_ref[r0:r0 + nr, h * DH:(h + 1) * DH]
            s0 = lax.dot_general(qh, k0_buf[h], (((1,), (1,)), ((), ())),
                                 preferred_element_type=jnp.float32)
            p0 = jnp.exp(s0 + bias0[r0:r0 + nr, :])
            den = jnp.sum(p0, axis=1, keepdims=True)
            acc = lax.dot_general(p0.astype(jnp.bfloat16), v0_buf[h],
                                  (((1,), (0,)), ((), ())),
                                  preferred_element_type=jnp.float32)
            s1 = lax.dot_general(qh, k1_buf[h], (((1,), (1,)), ((), ())),
                                 preferred_element_type=jnp.float32)
            p1 = jnp.exp(s1 + bias1[r0:r0 + nr, :])
            den = den + jnp.sum(p1, axis=1, keepdims=True)
            acc = acc + lax.dot_general(p1.astype(jnp.bfloat16), v1_buf[h],
                                        (((1,), (0,)), ((), ())),
                                        preferred_element_type=jnp.float32)
            band_ctx = (acc / den).astype(jnp.bfloat16)
            po = jnp.dot(band_ctx, wo_ref[h * DH:(h + 1) * DH, :],
                         preferred_element_type=jnp.float32)
            if h == 0:
                out_ref[r0:r0 + nr, :] = po
            else:
                out_ref[r0:r0 + nr, :] = out_ref[r0:r0 + nr, :] + po

    _band_pass(32, HALF - 32)

    pltpu.make_async_copy(ga_send.at[my], ga_recv.at[my], cp_sem.at[5]).wait()
    pltpu.make_async_copy(gd_send.at[my], gd_recv.at[my], cp_sem.at[6]).wait()
    for d in range(1, N_DEV):
        src_dev = (my - d) % N_DEV
        for t, (src, dst) in enumerate(((ga_send, ga_recv),
                                        (gd_send, gd_recv))):
            pltpu.make_async_remote_copy(
                src_ref=src.at[src_dev], dst_ref=dst.at[src_dev],
                send_sem=ga_send_s.at[0, t], recv_sem=ga_recv_s.at[src_dev, t],
                device_id=(my,), device_id_type=_MESH).wait_recv()

    for h in range(HQ):
        acc = jnp.zeros((NG, DH), jnp.float32)
        den = jnp.zeros((NG, 1), jnp.float32)
        for j in range(N_DEV):
            acc = acc + ga_recv[j, h].astype(jnp.float32)
            den = den + gd_recv[j, h]
        gctx = (acc / den).astype(jnp.bfloat16)
        po = jnp.dot(gctx, wo_ref[h * DH:(h + 1) * DH, :],
                     preferred_element_type=jnp.float32)
        if h == 0:
            out_ref[0:32, :] = po
        else:
            out_ref[0:32, :] = out_ref[0:32, :] + po

    def _bf_exchange(stage, half, r0):
        partner = jnp.bitwise_xor(my, stage + 1)
        sbuf[stage, r0:r0 + HALF, :] = (
            out_ref[r0:r0 + HALF, :].astype(jnp.bfloat16))
        ex = pltpu.make_async_remote_copy(
            src_ref=sbuf.at[stage, pl.ds(r0, HALF)],
            dst_ref=rbuf.at[stage, pl.ds(r0, HALF)],
            send_sem=bf_send.at[stage, half],
            recv_sem=bf_recv.at[stage, half],
            device_id=(partner,), device_id_type=_MESH)
        ex.start()
        return ex

    def _bf_finish(ex, stage, r0):
        ex.wait()
        out_ref[r0:r0 + HALF, :] = (
            out_ref[r0:r0 + HALF, :]
            + rbuf[stage, r0:r0 + HALF, :].astype(jnp.float32))

    s1a = _bf_exchange(0, 0, 0)
    _band_pass(HALF, HALF)
    s1b = _bf_exchange(0, 1, HALF)
    _bf_finish(s1a, 0, 0)
    s2a = _bf_exchange(1, 0, 0)
    _bf_finish(s1b, 0, HALF)
    s2b = _bf_exchange(1, 1, HALF)
    _bf_finish(s2a, 1, 0)
    _bf_finish(s2b, 1, HALF)

    for d in range(1, N_DEV):
        peer = (my + d) % N_DEV
        pltpu.make_async_remote_copy(
            src_ref=q_ref.at[pl.ds(0, NG)], dst_ref=qg_buf.at[my],
            send_sem=qg_send_s.at[d - 1], recv_sem=qg_recv_s.at[my],
            device_id=(peer,), device_id_type=_MESH).wait_send()
        for t, (src, dst) in enumerate(((ga_send, ga_recv),
                                        (gd_send, gd_recv))):
            pltpu.make_async_remote_copy(
                src_ref=src.at[peer], dst_ref=dst.at[my],
                send_sem=ga_send_s.at[d - 1, t], recv_sem=ga_recv_s.at[my, t],
                device_id=(peer,), device_id_type=_MESH).wait_send()

    @pl.when(my == 0)
    def _():
        for i, d in enumerate((1, 3)):
            peer = (my + d) % N_DEV
            for t, (src, dst) in enumerate(kv_pairs):
                pltpu.make_async_remote_copy(
                    src_ref=src.at[pl.ds(peer * HQ, HQ)], dst_ref=dst,
                    send_sem=k0_send.at[i, t], recv_sem=k0_recv.at[t],
                    device_id=(peer,), device_id_type=_MESH).wait_send()

    @pl.when(my == 1)
    def _():
        for d in range(1, N_DEV):
            peer = (my + d) % N_DEV
            for t, (src, dst) in enumerate(((kt_ref, k1_buf),
                                            (vt_ref, v1_buf))):
                pltpu.make_async_remote_copy(
                    src_ref=src.at[pl.ds(peer * HQ, HQ), pl.ds(0, NB1)],
                    dst_ref=dst,
                    send_sem=k1_send.at[d - 1, t], recv_sem=k1_recv.at[t],
                    device_id=(peer,), device_id_type=_MESH).wait_send()
        pltpu.make_async_remote_copy(
            src_ref=relay_buf, dst_ref=k0_buf,
            send_sem=fwd_send.at[0], recv_sem=k0_recv.at[0],
            device_id=((my + 1) % N_DEV,), device_id_type=_MESH).wait_send()

    @pl.when(my == 3)
    def _():
        pltpu.make_async_remote_copy(
            src_ref=relay_buf, dst_ref=v0_buf,
            send_sem=fwd_send.at[1], recv_sem=k0_recv.at[1],
            device_id=((my - 1) % N_DEV,), device_id_type=_MESH).wait_send()

    def _exit(sem):
        for d in range(1, N_DEV):
            pl.semaphore_signal(sem, inc=1, device_id=((my + d) % N_DEV,),
                                device_id_type=_MESH)
        pl.semaphore_wait(sem, N_DEV - 1)
    pl.run_scoped(_exit, sem=pltpu.SemaphoreType.REGULAR)


def kernel(x, Wq, K_ext, V_ext, Wo):
    kt = K_ext[0].transpose(1, 0, 2).astype(jnp.bfloat16)
    vt = V_ext[0].transpose(1, 0, 2).astype(jnp.bfloat16)
    q = (jnp.dot(x[0], Wq, preferred_element_type=jnp.float32)
         * SCALE).astype(jnp.bfloat16)
    wo = Wo.astype(jnp.bfloat16)

    out = pl.pallas_call(
        _body,
        out_shape=jax.ShapeDtypeStruct((SQ, DM), jnp.float32),
        in_specs=[
            pl.BlockSpec(memory_space=pltpu.VMEM),
            pl.BlockSpec(memory_space=pltpu.VMEM),
            pl.BlockSpec(memory_space=pltpu.VMEM),
            pl.BlockSpec(memory_space=pltpu.VMEM),
        ],
        out_specs=pl.BlockSpec(memory_space=pltpu.VMEM),
        scratch_shapes=[
            pltpu.VMEM((HQ, SKV, DH), jnp.bfloat16),
            pltpu.VMEM((HQ, SKV, DH), jnp.bfloat16),
            pltpu.VMEM((HQ, NB1, DH), jnp.bfloat16),
            pltpu.VMEM((HQ, NB1, DH), jnp.bfloat16),
            pltpu.VMEM((HQ, SKV, DH), jnp.bfloat16),
            pltpu.VMEM((N_DEV, NG, DM), jnp.bfloat16),
            pltpu.VMEM((N_DEV, HQ, NG, DH), jnp.bfloat16),
            pltpu.VMEM((N_DEV, HQ, NG, 1), jnp.float32),
            pltpu.VMEM((N_DEV, HQ, NG, DH), jnp.bfloat16),
            pltpu.VMEM((N_DEV, HQ, NG, 1), jnp.float32),
            pltpu.VMEM((2, SQ, DM), jnp.bfloat16),
            pltpu.VMEM((2, SQ, DM), jnp.bfloat16),
            pltpu.SemaphoreType.DMA((2, 2)),
            pltpu.SemaphoreType.DMA((2,)),
            pltpu.SemaphoreType.DMA((2,)),
            pltpu.SemaphoreType.DMA((2,)),
            pltpu.SemaphoreType.DMA((2,)),
            pltpu.SemaphoreType.DMA((N_DEV - 1, 2)),
            pltpu.SemaphoreType.DMA((2,)),
            pltpu.SemaphoreType.DMA((N_DEV - 1,)),
            pltpu.SemaphoreType.DMA((N_DEV,)),
            pltpu.SemaphoreType.DMA((N_DEV - 1, 2)),
            pltpu.SemaphoreType.DMA((N_DEV, 2)),
            pltpu.SemaphoreType.DMA((7,)),
            pltpu.SemaphoreType.DMA((2, 2)),
            pltpu.SemaphoreType.DMA((2, 2)),
        ],
        compiler_params=pltpu.CompilerParams(
            collective_id=0, vmem_limit_bytes=46 * 1024 * 1024),
    )(q, wo, kt, vt)
    return out[None]
